# baseline (speedup 1.0000x reference)
import functools

import jax
import jax.numpy as jnp
from jax import lax
from jax.experimental import pallas as pl
from jax.experimental.pallas import tpu as pltpu

K_TOP = 100
K_LR = 0.1


def _ordered_key_u32(s):
    u = lax.bitcast_convert_type(s, jnp.uint32)
    flip = jnp.where(u >= jnp.uint32(0x80000000),
                     jnp.uint32(0xFFFFFFFF), jnp.uint32(0x80000000))
    return u ^ flip


def _fused_body(scores_ref, at_ref, oldloc_ref, oldscale_ref, out_ref,
                thr_ref, idxthr_ref, acc_ref, accsq_ref):
    ni = pl.program_id(1)
    nn = pl.num_programs(1)
    nb = at_ref.shape[0]

    @pl.when(ni == 0)
    def _thresholds():
        s = scores_ref[...]
        n, bb = s.shape
        key = _ordered_key_u32(s)
        ones = jnp.ones((8, n), dtype=jnp.float32)

        def count(mask):
            mf = mask.astype(jnp.float32)
            return jax.lax.dot_general(
                ones, mf, (((1,), (0,)), ((), ())),
                preferred_element_type=jnp.float32)[0]

        ktop = jnp.float32(K_TOP)
        p = jnp.zeros((bb,), dtype=jnp.uint32)
        for bit in range(31, -1, -1):
            cand = p | jnp.uint32(1 << bit)
            cnt = count(key >= cand[None, :])
            p = jnp.where(cnt >= ktop, cand, p)

        need = ktop - count(key > p[None, :])

        eq = key == p[None, :]
        iota = lax.broadcasted_iota(jnp.int32, (n, bb), 0)
        lo = jnp.zeros((bb,), dtype=jnp.int32)
        hi = jnp.full((bb,), n, dtype=jnp.int32)
        for _ in range(10):
            mid = (lo + hi) // 2
            c = count(eq & (iota < mid[None, :]))
            cond = c >= need
            hi = jnp.where(cond, mid, hi)
            lo = jnp.where(cond, lo, mid)

        thr_ref[...] = p
        idxthr_ref[...] = hi
        acc_ref[...] = jnp.zeros_like(acc_ref)
        accsq_ref[...] = jnp.zeros_like(accsq_ref)

    s = scores_ref[pl.ds(ni * nb, nb), :]
    bb = s.shape[1]
    key = _ordered_key_u32(s)
    thr = thr_ref[...][None, :]
    idxthr = idxthr_ref[...][None, :]
    iota = ni * nb + lax.broadcasted_iota(jnp.int32, (nb, bb), 0)
    m = (key > thr) | ((key == thr) & (iota < idxthr))
    mf = m.astype(jnp.float32)

    a = at_ref[...]
    am = a * mf[:, None, :]
    acc_ref[...] += jnp.sum(am, axis=0)
    accsq_ref[...] += jnp.sum(am * am, axis=0)

    @pl.when(ni == nn - 1)
    def _finalize():
        tot = acc_ref[...]
        totsq = accsq_ref[...]
        mean = tot * (1.0 / K_TOP)
        var = (totsq - tot * mean) * (1.0 / (K_TOP - 1))
        scale = jnp.sqrt(var + 1e-6)
        new_loc = (1.0 - K_LR) * oldloc_ref[...] + K_LR * mean
        new_scale = (1.0 - K_LR) * oldscale_ref[...] + K_LR * scale
        out_ref[...] = jnp.stack([new_loc, new_scale], axis=0)


@jax.jit
def kernel(obj, actions, old_loc, old_scale):
    N, B, V = actions.shape
    scores = obj[..., 0]
    at = jnp.moveaxis(actions, -1, 1)
    oldloc_t = old_loc.T
    oldscale_t = old_scale.T
    BB = 512
    NB = 128

    out_t = pl.pallas_call(
        _fused_body,
        grid=(B // BB, N // NB),
        in_specs=[
            pl.BlockSpec((N, BB), lambda bi, ni: (0, bi)),
            pl.BlockSpec((NB, V, BB), lambda bi, ni: (ni, 0, bi)),
            pl.BlockSpec((V, BB), lambda bi, ni: (0, bi)),
            pl.BlockSpec((V, BB), lambda bi, ni: (0, bi)),
        ],
        out_specs=pl.BlockSpec((2, V, BB), lambda bi, ni: (0, 0, bi)),
        out_shape=jax.ShapeDtypeStruct((2, V, B), jnp.float32),
        scratch_shapes=[pltpu.VMEM((BB,), jnp.uint32),
                        pltpu.VMEM((BB,), jnp.int32),
                        pltpu.VMEM((V, BB), jnp.float32),
                        pltpu.VMEM((V, BB), jnp.float32)],
    )(scores, at, oldloc_t, oldscale_t)

    return jnp.moveaxis(out_t, 1, -1)


# speedup vs baseline: 1.1165x; 1.1165x over previous
import functools

import jax
import jax.numpy as jnp
from jax import lax
from jax.experimental import pallas as pl
from jax.experimental.pallas import tpu as pltpu

K_TOP = 100
K_LR = 0.1


def _ordered_key_u32(s):
    u = lax.bitcast_convert_type(s, jnp.uint32)
    flip = jnp.where(u >= jnp.uint32(0x80000000),
                     jnp.uint32(0xFFFFFFFF), jnp.uint32(0x80000000))
    return u ^ flip


def _fused_body(scores_ref, at_ref, oldloc_ref, oldscale_ref, out_ref,
                thr_ref, idxthr_ref, acc_ref, accsq_ref):
    ni = pl.program_id(1)
    nn = pl.num_programs(1)
    nb = at_ref.shape[0]

    @pl.when(ni == 0)
    def _thresholds():
        s = scores_ref[...]
        n, bb = s.shape
        key = _ordered_key_u32(s)

        kh = (key >> 16).astype(jnp.int32) - 32768
        kh16 = kh.astype(jnp.int16)
        kl = (key & jnp.uint32(0xFFFF)).astype(jnp.int32) - 32768
        kl16 = kl.astype(jnp.int16)

        def cnt16(mask):
            m = mask.astype(jnp.int16)
            rows = m.shape[0]
            while rows > 16:
                half = rows // 2
                m = m[:half] + m[half:rows]
                rows = half
            return jnp.sum(m.astype(jnp.int32), axis=0)

        ph = jnp.full((bb,), -32768, dtype=jnp.int32)
        for bit in range(15, -1, -1):
            cand = ph + (1 << bit)
            c = cnt16(kh16 >= cand.astype(jnp.int16)[None, :])
            ph = jnp.where(c >= K_TOP, cand, ph)
        ph16 = ph.astype(jnp.int16)

        band = kh16 == ph16[None, :]
        c_above = cnt16(kh16 > ph16[None, :])
        pl_ = jnp.full((bb,), -32768, dtype=jnp.int32)
        for bit in range(15, -1, -1):
            cand = pl_ + (1 << bit)
            c = cnt16(band & (kl16 >= cand.astype(jnp.int16)[None, :]))
            pl_ = jnp.where(c_above + c >= K_TOP, cand, pl_)
        pl16 = pl_.astype(jnp.int16)

        eq = band & (kl16 == pl16[None, :])
        need = K_TOP - c_above - cnt16(band & (kl16 > pl16[None, :]))

        iota = lax.broadcasted_iota(jnp.int16, (n, bb), 0)
        lo = jnp.zeros((bb,), dtype=jnp.int32)
        hi = jnp.full((bb,), n, dtype=jnp.int32)
        for _ in range(10):
            mid = (lo + hi) >> 1
            c = cnt16(eq & (iota < mid.astype(jnp.int16)[None, :]))
            cond = c >= need
            hi = jnp.where(cond, mid, hi)
            lo = jnp.where(cond, lo, mid)

        thr = ((ph + 32768).astype(jnp.uint32) << 16) | \
            (pl_ + 32768).astype(jnp.uint32)
        thr_ref[...] = thr
        idxthr_ref[...] = hi
        acc_ref[...] = jnp.zeros_like(acc_ref)
        accsq_ref[...] = jnp.zeros_like(accsq_ref)

    s = scores_ref[pl.ds(ni * nb, nb), :]
    bb = s.shape[1]
    key = _ordered_key_u32(s)
    thr = thr_ref[...][None, :]
    idxthr = idxthr_ref[...][None, :]
    iota = ni * nb + lax.broadcasted_iota(jnp.int32, (nb, bb), 0)
    m = (key > thr) | ((key == thr) & (iota < idxthr))
    mf = m.astype(jnp.float32)

    a = at_ref[...]
    am = a * mf[:, None, :]
    acc_ref[...] += jnp.sum(am, axis=0)
    accsq_ref[...] += jnp.sum(am * am, axis=0)

    @pl.when(ni == nn - 1)
    def _finalize():
        tot = acc_ref[...]
        totsq = accsq_ref[...]
        mean = tot * (1.0 / K_TOP)
        var = (totsq - tot * mean) * (1.0 / (K_TOP - 1))
        scale = jnp.sqrt(var + 1e-6)
        new_loc = (1.0 - K_LR) * oldloc_ref[...] + K_LR * mean
        new_scale = (1.0 - K_LR) * oldscale_ref[...] + K_LR * scale
        out_ref[...] = jnp.stack([new_loc, new_scale], axis=0)


@jax.jit
def kernel(obj, actions, old_loc, old_scale):
    N, B, V = actions.shape
    scores = obj[..., 0]
    at = jnp.moveaxis(actions, -1, 1)
    oldloc_t = old_loc.T
    oldscale_t = old_scale.T
    BB = 512
    NB = 128

    out_t = pl.pallas_call(
        _fused_body,
        grid=(B // BB, N // NB),
        in_specs=[
            pl.BlockSpec((N, BB), lambda bi, ni: (0, bi)),
            pl.BlockSpec((NB, V, BB), lambda bi, ni: (ni, 0, bi)),
            pl.BlockSpec((V, BB), lambda bi, ni: (0, bi)),
            pl.BlockSpec((V, BB), lambda bi, ni: (0, bi)),
        ],
        out_specs=pl.BlockSpec((2, V, BB), lambda bi, ni: (0, 0, bi)),
        out_shape=jax.ShapeDtypeStruct((2, V, B), jnp.float32),
        scratch_shapes=[pltpu.VMEM((BB,), jnp.uint32),
                        pltpu.VMEM((BB,), jnp.int32),
                        pltpu.VMEM((V, BB), jnp.float32),
                        pltpu.VMEM((V, BB), jnp.float32)],
    )(scores, at, oldloc_t, oldscale_t)

    return jnp.moveaxis(out_t, 1, -1)


# speedup vs baseline: 1.1488x; 1.0290x over previous
import functools

import jax
import jax.numpy as jnp
from jax import lax
from jax.experimental import pallas as pl
from jax.experimental.pallas import tpu as pltpu

K_TOP = 100
K_LR = 0.1


def _ordered_key_u32(s):
    u = lax.bitcast_convert_type(s, jnp.uint32)
    flip = jnp.where(u >= jnp.uint32(0x80000000),
                     jnp.uint32(0xFFFFFFFF), jnp.uint32(0x80000000))
    return u ^ flip


def _fused_body(scores_ref, at_ref, oldloc_ref, oldscale_ref, out_ref,
                thr_ref, idxthr_ref, acc_ref, accsq_ref):
    ni = pl.program_id(1)
    nn = pl.num_programs(1)
    nb = at_ref.shape[0]

    @pl.when(ni == 0)
    def _thresholds():
        s = scores_ref[...]
        n, bb = s.shape
        key = _ordered_key_u32(s)

        kh = (key >> 16).astype(jnp.int32) - 32768
        kh16 = kh.astype(jnp.int16)
        kl = (key & jnp.uint32(0xFFFF)).astype(jnp.int32) - 32768
        kl16 = kl.astype(jnp.int16)

        def cnt16(mask):
            m = mask.astype(jnp.int16)
            rows = m.shape[0]
            while rows > 16:
                half = rows // 2
                m = m[:half] + m[half:rows]
                rows = half
            return jnp.sum(m.astype(jnp.int32), axis=0)

        ph = jnp.full((bb,), -32768, dtype=jnp.int32)
        for bit in range(15, -1, -1):
            cand = ph + (1 << bit)
            c = cnt16(kh16 >= cand.astype(jnp.int16)[None, :])
            ph = jnp.where(c >= K_TOP, cand, ph)
        ph16 = ph.astype(jnp.int16)

        band = kh16 == ph16[None, :]
        c_above = cnt16(kh16 > ph16[None, :])
        pl_ = jnp.full((bb,), -32768, dtype=jnp.int32)
        for bit in range(15, -1, -1):
            cand = pl_ + (1 << bit)
            c = cnt16(band & (kl16 >= cand.astype(jnp.int16)[None, :]))
            pl_ = jnp.where(c_above + c >= K_TOP, cand, pl_)
        pl16 = pl_.astype(jnp.int16)

        eq = band & (kl16 == pl16[None, :])
        need = K_TOP - c_above - cnt16(band & (kl16 > pl16[None, :]))

        iota = lax.broadcasted_iota(jnp.int16, (n, bb), 0)
        lo = jnp.zeros((bb,), dtype=jnp.int32)
        hi = jnp.full((bb,), n, dtype=jnp.int32)
        for _ in range(10):
            mid = (lo + hi) >> 1
            c = cnt16(eq & (iota < mid.astype(jnp.int16)[None, :]))
            cond = c >= need
            hi = jnp.where(cond, mid, hi)
            lo = jnp.where(cond, lo, mid)

        thr = ((ph + 32768).astype(jnp.uint32) << 16) | \
            (pl_ + 32768).astype(jnp.uint32)
        thr_ref[...] = thr
        idxthr_ref[...] = hi
        acc_ref[...] = jnp.zeros_like(acc_ref)
        accsq_ref[...] = jnp.zeros_like(accsq_ref)

    s = scores_ref[pl.ds(ni * nb, nb), :]
    bb = s.shape[1]
    key = _ordered_key_u32(s)
    thr = thr_ref[...][None, :]
    idxthr = idxthr_ref[...][None, :]
    iota = ni * nb + lax.broadcasted_iota(jnp.int32, (nb, bb), 0)
    m = (key > thr) | ((key == thr) & (iota < idxthr))
    mf = m.astype(jnp.float32)

    a = at_ref[...]
    am = a * mf[:, None, :]
    acc_ref[...] += jnp.sum(am, axis=0)
    accsq_ref[...] += jnp.sum(am * am, axis=0)

    @pl.when(ni == nn - 1)
    def _finalize():
        tot = acc_ref[...]
        totsq = accsq_ref[...]
        mean = tot * (1.0 / K_TOP)
        var = (totsq - tot * mean) * (1.0 / (K_TOP - 1))
        scale = jnp.sqrt(var + 1e-6)
        new_loc = (1.0 - K_LR) * oldloc_ref[...] + K_LR * mean
        new_scale = (1.0 - K_LR) * oldscale_ref[...] + K_LR * scale
        out_ref[...] = jnp.stack([new_loc, new_scale], axis=0)


@jax.jit
def kernel(obj, actions, old_loc, old_scale):
    N, B, V = actions.shape
    scores = obj[..., 0]
    at = jnp.moveaxis(actions, -1, 1)
    oldloc_t = old_loc.T
    oldscale_t = old_scale.T
    BB = 1024
    NB = 128

    out_t = pl.pallas_call(
        _fused_body,
        grid=(B // BB, N // NB),
        in_specs=[
            pl.BlockSpec((N, BB), lambda bi, ni: (0, bi)),
            pl.BlockSpec((NB, V, BB), lambda bi, ni: (ni, 0, bi)),
            pl.BlockSpec((V, BB), lambda bi, ni: (0, bi)),
            pl.BlockSpec((V, BB), lambda bi, ni: (0, bi)),
        ],
        out_specs=pl.BlockSpec((2, V, BB), lambda bi, ni: (0, 0, bi)),
        out_shape=jax.ShapeDtypeStruct((2, V, B), jnp.float32),
        scratch_shapes=[pltpu.VMEM((BB,), jnp.uint32),
                        pltpu.VMEM((BB,), jnp.int32),
                        pltpu.VMEM((V, BB), jnp.float32),
                        pltpu.VMEM((V, BB), jnp.float32)],
    )(scores, at, oldloc_t, oldscale_t)

    return jnp.moveaxis(out_t, 1, -1)
